# baseline (speedup 1.0000x reference)
import functools

import jax
import jax.numpy as jnp
from jax.experimental import pallas as pl

B = 32768
DS = 658
DB = 250
H = 256
A = 20
BM = 1024
OUT_W = 128


def _body(xs_ref, xb_ref, act_ref, ws_ref, wb_ref, b1_ref, w2_ref, b2_ref,
          lp_ref, val_ref, ent_ref):
    bf = jnp.bfloat16
    acc = jnp.dot(xs_ref[...].astype(bf), ws_ref[...].astype(bf),
                  preferred_element_type=jnp.float32)
    acc = acc + jnp.dot(xb_ref[...].astype(bf), wb_ref[...].astype(bf),
                        preferred_element_type=jnp.float32)
    h = jnp.tanh(acc + b1_ref[...])
    o2 = jnp.dot(h.astype(bf), w2_ref[...].astype(bf),
                 preferred_element_type=jnp.float32) + b2_ref[...]
    logits = o2[:, :A]
    value = o2[:, A]
    m = jnp.max(logits, axis=-1, keepdims=True)
    e = jnp.exp(logits - m)
    z = jnp.sum(e, axis=-1, keepdims=True)
    logp = logits - m - jnp.log(z)
    ent = -jnp.sum((e / z) * logp, axis=-1)
    onehot = jax.lax.broadcasted_iota(jnp.int32, logits.shape, 1) == act_ref[...]
    alp = jnp.sum(jnp.where(onehot, logp, 0.0), axis=-1)
    lp_ref[...] = alp[:, None]
    val_ref[...] = value[:, None]
    ent_ref[...] = ent[:, None]


@functools.partial(jax.jit, static_argnames=("interpret",))
def _run(states, believes, actions, Wa1, ba1, Wa2, ba2, Wc1, bc1, Wc2, bc2,
         interpret=False):
    ws = jnp.concatenate([Wa1[:DS], Wc1], axis=1)
    wb = jnp.concatenate([Wa1[DS:], jnp.zeros((DB, H), jnp.float32)], axis=1)
    b1 = jnp.concatenate([ba1, bc1])[None, :]
    w2 = jnp.zeros((2 * H, OUT_W), jnp.float32)
    w2 = w2.at[:H, :A].set(Wa2).at[H:, A].set(Wc2[:, 0])
    b2 = jnp.zeros((OUT_W,), jnp.float32).at[:A].set(ba2).at[A].set(bc2[0])
    b2 = b2[None, :]
    act2d = actions.astype(jnp.int32)[:, None]

    grid = (B // BM,)
    out = pl.pallas_call(
        _body,
        grid=grid,
        in_specs=[
            pl.BlockSpec((BM, DS), lambda i: (i, 0)),
            pl.BlockSpec((BM, DB), lambda i: (i, 0)),
            pl.BlockSpec((BM, 1), lambda i: (i, 0)),
            pl.BlockSpec((DS, 2 * H), lambda i: (0, 0)),
            pl.BlockSpec((DB, 2 * H), lambda i: (0, 0)),
            pl.BlockSpec((1, 2 * H), lambda i: (0, 0)),
            pl.BlockSpec((2 * H, OUT_W), lambda i: (0, 0)),
            pl.BlockSpec((1, OUT_W), lambda i: (0, 0)),
        ],
        out_specs=[
            pl.BlockSpec((BM, 1), lambda i: (i, 0)),
            pl.BlockSpec((BM, 1), lambda i: (i, 0)),
            pl.BlockSpec((BM, 1), lambda i: (i, 0)),
        ],
        out_shape=[jax.ShapeDtypeStruct((B, 1), jnp.float32)] * 3,
        interpret=interpret,
    )(states, believes, act2d, ws, wb, b1, w2, b2)
    return out[0][:, 0], out[1][:, 0], out[2][:, 0]


def kernel(states, believes, actions, Wa1, ba1, Wa2, ba2, Wc1, bc1, Wc2, bc2):
    return _run(states, believes, actions, Wa1, ba1, Wa2, ba2,
                Wc1, bc1, Wc2, bc2)


# speedup vs baseline: 1.5764x; 1.5764x over previous
import functools

import jax
import jax.numpy as jnp
from jax.experimental import pallas as pl

B = 32768
DS = 658
DB = 250
H = 256
A = 20
BM = 1024
OUT_W = 128
_MM_DTYPE = jnp.bfloat16


def _body(xs_ref, xb_ref, act_ref, ws_ref, wb_ref, b1_ref, w2_ref, b2_ref,
          lp_ref, val_ref, ent_ref):
    bf = _MM_DTYPE
    acc = jnp.dot(xs_ref[...].astype(bf), ws_ref[...].astype(bf),
                  preferred_element_type=jnp.float32)
    acc = acc + jnp.dot(xb_ref[...].astype(bf), wb_ref[...].astype(bf),
                        preferred_element_type=jnp.float32)
    h = jnp.tanh(acc + b1_ref[...])
    o2 = jnp.dot(h.astype(bf), w2_ref[...].astype(bf),
                 preferred_element_type=jnp.float32)
    o2t = jnp.swapaxes(o2, 0, 1) + b2_ref[...]
    logits = o2t[:A]
    value = o2t[A:A + 1]
    m = jnp.max(logits, axis=0, keepdims=True)
    e = jnp.exp(logits - m)
    z = jnp.sum(e, axis=0, keepdims=True)
    logz = jnp.log(z)
    s = jnp.sum(e * (logits - m), axis=0, keepdims=True)
    ent = logz - s / z
    onehot = jax.lax.broadcasted_iota(jnp.int32, logits.shape, 0) == act_ref[...]
    g = jnp.sum(jnp.where(onehot, logits, 0.0), axis=0, keepdims=True)
    alp = g - m - logz
    lp_ref[...] = alp
    val_ref[...] = value
    ent_ref[...] = ent


@functools.partial(jax.jit, static_argnames=("interpret",))
def _run(states, believes, actions, Wa1, ba1, Wa2, ba2, Wc1, bc1, Wc2, bc2,
         interpret=False):
    ws = jnp.concatenate([Wa1[:DS], Wc1], axis=1)
    wb = jnp.concatenate([Wa1[DS:], jnp.zeros((DB, H), jnp.float32)], axis=1)
    b1 = jnp.concatenate([ba1, bc1])[None, :]
    w2 = jnp.zeros((2 * H, OUT_W), jnp.float32)
    w2 = w2.at[:H, :A].set(Wa2).at[H:, A].set(Wc2[:, 0])
    b2 = jnp.zeros((OUT_W, 1), jnp.float32).at[:A, 0].set(ba2).at[A, 0].set(bc2[0])
    act2d = actions.astype(jnp.int32)[None, :]

    grid = (B // BM,)
    out = pl.pallas_call(
        _body,
        grid=grid,
        in_specs=[
            pl.BlockSpec((BM, DS), lambda i: (i, 0)),
            pl.BlockSpec((BM, DB), lambda i: (i, 0)),
            pl.BlockSpec((1, BM), lambda i: (0, i)),
            pl.BlockSpec((DS, 2 * H), lambda i: (0, 0)),
            pl.BlockSpec((DB, 2 * H), lambda i: (0, 0)),
            pl.BlockSpec((1, 2 * H), lambda i: (0, 0)),
            pl.BlockSpec((2 * H, OUT_W), lambda i: (0, 0)),
            pl.BlockSpec((OUT_W, 1), lambda i: (0, 0)),
        ],
        out_specs=[
            pl.BlockSpec((1, BM), lambda i: (0, i)),
            pl.BlockSpec((1, BM), lambda i: (0, i)),
            pl.BlockSpec((1, BM), lambda i: (0, i)),
        ],
        out_shape=[jax.ShapeDtypeStruct((1, B), jnp.float32)] * 3,
        interpret=interpret,
    )(states, believes, act2d, ws, wb, b1, w2, b2)
    return out[0][0], out[1][0], out[2][0]


def kernel(states, believes, actions, Wa1, ba1, Wa2, ba2, Wc1, bc1, Wc2, bc2):
    return _run(states, believes, actions, Wa1, ba1, Wa2, ba2,
                Wc1, bc1, Wc2, bc2)


# speedup vs baseline: 1.6784x; 1.0647x over previous
import functools

import jax
import jax.numpy as jnp
from jax.experimental import pallas as pl

B = 32768
DS = 658
DB = 250
H = 256
A = 20
BM = 2048
OUT_W = 128
_MM_DTYPE = jnp.bfloat16


def _body(xs_ref, xb_ref, act_ref, ws_ref, wb_ref, b1_ref, w2_ref, b2_ref,
          lp_ref, val_ref, ent_ref):
    bf = _MM_DTYPE
    acc = jnp.dot(xs_ref[...].astype(bf), ws_ref[...].astype(bf),
                  preferred_element_type=jnp.float32)
    acc = acc + jnp.dot(xb_ref[...].astype(bf), wb_ref[...].astype(bf),
                        preferred_element_type=jnp.float32)
    h = jnp.tanh(acc + b1_ref[...])
    o2 = jnp.dot(h.astype(bf), w2_ref[...].astype(bf),
                 preferred_element_type=jnp.float32)
    o2t = jnp.swapaxes(o2, 0, 1) + b2_ref[...]
    logits = o2t[:A]
    value = o2t[A:A + 1]
    m = jnp.max(logits, axis=0, keepdims=True)
    e = jnp.exp(logits - m)
    z = jnp.sum(e, axis=0, keepdims=True)
    logz = jnp.log(z)
    s = jnp.sum(e * (logits - m), axis=0, keepdims=True)
    ent = logz - s / z
    onehot = jax.lax.broadcasted_iota(jnp.int32, logits.shape, 0) == act_ref[...]
    g = jnp.sum(jnp.where(onehot, logits, 0.0), axis=0, keepdims=True)
    alp = g - m - logz
    lp_ref[...] = alp
    val_ref[...] = value
    ent_ref[...] = ent


@functools.partial(jax.jit, static_argnames=("interpret",))
def _run(states, believes, actions, Wa1, ba1, Wa2, ba2, Wc1, bc1, Wc2, bc2,
         interpret=False):
    ws = jnp.concatenate([Wa1[:DS], Wc1], axis=1)
    wb = jnp.concatenate([Wa1[DS:], jnp.zeros((DB, H), jnp.float32)], axis=1)
    b1 = jnp.concatenate([ba1, bc1])[None, :]
    w2 = jnp.zeros((2 * H, OUT_W), jnp.float32)
    w2 = w2.at[:H, :A].set(Wa2).at[H:, A].set(Wc2[:, 0])
    b2 = jnp.zeros((OUT_W, 1), jnp.float32).at[:A, 0].set(ba2).at[A, 0].set(bc2[0])
    act2d = actions.astype(jnp.int32)[None, :]

    grid = (B // BM,)
    out = pl.pallas_call(
        _body,
        grid=grid,
        in_specs=[
            pl.BlockSpec((BM, DS), lambda i: (i, 0)),
            pl.BlockSpec((BM, DB), lambda i: (i, 0)),
            pl.BlockSpec((1, BM), lambda i: (0, i)),
            pl.BlockSpec((DS, 2 * H), lambda i: (0, 0)),
            pl.BlockSpec((DB, 2 * H), lambda i: (0, 0)),
            pl.BlockSpec((1, 2 * H), lambda i: (0, 0)),
            pl.BlockSpec((2 * H, OUT_W), lambda i: (0, 0)),
            pl.BlockSpec((OUT_W, 1), lambda i: (0, 0)),
        ],
        out_specs=[
            pl.BlockSpec((1, BM), lambda i: (0, i)),
            pl.BlockSpec((1, BM), lambda i: (0, i)),
            pl.BlockSpec((1, BM), lambda i: (0, i)),
        ],
        out_shape=[jax.ShapeDtypeStruct((1, B), jnp.float32)] * 3,
        interpret=interpret,
    )(states, believes, act2d, ws, wb, b1, w2, b2)
    return out[0][0], out[1][0], out[2][0]


def kernel(states, believes, actions, Wa1, ba1, Wa2, ba2, Wc1, bc1, Wc2, bc2):
    return _run(states, believes, actions, Wa1, ba1, Wa2, ba2,
                Wc1, bc1, Wc2, bc2)


# speedup vs baseline: 1.6807x; 1.0014x over previous
import functools

import jax
import jax.numpy as jnp
from jax.experimental import pallas as pl
from jax.experimental.pallas import tpu as pltpu

B = 32768
DS = 658
DB = 250
H = 256
A = 20
BM = 2048
OUT_W = 128
_MM_DTYPE = jnp.bfloat16


def _body(xs_ref, xb_ref, act_ref, ws_ref, wb_ref, b1_ref, w2_ref, b2_ref,
          lp_ref, val_ref, ent_ref):
    bf = _MM_DTYPE
    acc = jnp.dot(xs_ref[...].astype(bf), ws_ref[...].astype(bf),
                  preferred_element_type=jnp.float32)
    acc = acc + jnp.dot(xb_ref[...].astype(bf), wb_ref[...].astype(bf),
                        preferred_element_type=jnp.float32)
    h = jnp.tanh(acc + b1_ref[...])
    o2 = jnp.dot(h.astype(bf), w2_ref[...].astype(bf),
                 preferred_element_type=jnp.float32)
    o2t = jnp.swapaxes(o2, 0, 1) + b2_ref[...]
    logits = o2t[:A]
    value = o2t[A:A + 1]
    m = jnp.max(logits, axis=0, keepdims=True)
    e = jnp.exp(logits - m)
    z = jnp.sum(e, axis=0, keepdims=True)
    logz = jnp.log(z)
    s = jnp.sum(e * (logits - m), axis=0, keepdims=True)
    ent = logz - s / z
    onehot = jax.lax.broadcasted_iota(jnp.int32, logits.shape, 0) == act_ref[...]
    g = jnp.sum(jnp.where(onehot, logits, 0.0), axis=0, keepdims=True)
    alp = g - m - logz
    lp_ref[...] = alp
    val_ref[...] = value
    ent_ref[...] = ent


@functools.partial(jax.jit, static_argnames=("interpret",))
def _run(states, believes, actions, Wa1, ba1, Wa2, ba2, Wc1, bc1, Wc2, bc2,
         interpret=False):
    ws = jnp.concatenate([Wa1[:DS], Wc1], axis=1)
    wb = jnp.concatenate([Wa1[DS:], jnp.zeros((DB, H), jnp.float32)], axis=1)
    b1 = jnp.concatenate([ba1, bc1])[None, :]
    w2 = jnp.zeros((2 * H, OUT_W), jnp.float32)
    w2 = w2.at[:H, :A].set(Wa2).at[H:, A].set(Wc2[:, 0])
    b2 = jnp.zeros((OUT_W, 1), jnp.float32).at[:A, 0].set(ba2).at[A, 0].set(bc2[0])
    act2d = actions.astype(jnp.int32)[None, :]

    grid = (B // BM,)
    out = pl.pallas_call(
        _body,
        grid=grid,
        in_specs=[
            pl.BlockSpec((BM, DS), lambda i: (i, 0)),
            pl.BlockSpec((BM, DB), lambda i: (i, 0)),
            pl.BlockSpec((1, BM), lambda i: (0, i)),
            pl.BlockSpec((DS, 2 * H), lambda i: (0, 0)),
            pl.BlockSpec((DB, 2 * H), lambda i: (0, 0)),
            pl.BlockSpec((1, 2 * H), lambda i: (0, 0)),
            pl.BlockSpec((2 * H, OUT_W), lambda i: (0, 0)),
            pl.BlockSpec((OUT_W, 1), lambda i: (0, 0)),
        ],
        out_specs=[
            pl.BlockSpec((1, BM), lambda i: (0, i)),
            pl.BlockSpec((1, BM), lambda i: (0, i)),
            pl.BlockSpec((1, BM), lambda i: (0, i)),
        ],
        out_shape=[jax.ShapeDtypeStruct((1, B), jnp.float32)] * 3,
        compiler_params=pltpu.CompilerParams(
            dimension_semantics=("parallel",)),
        interpret=interpret,
    )(states, believes, act2d, ws, wb, b1, w2, b2)
    return out[0][0], out[1][0], out[2][0]


def kernel(states, believes, actions, Wa1, ba1, Wa2, ba2, Wc1, bc1, Wc2, bc2):
    return _run(states, believes, actions, Wa1, ba1, Wa2, ba2,
                Wc1, bc1, Wc2, bc2)


# speedup vs baseline: 4.1423x; 2.4646x over previous
import functools

import jax
import jax.numpy as jnp
from jax.experimental import pallas as pl
from jax.experimental.pallas import tpu as pltpu

B = 32768
DS = 658
DB = 250
H = 256
A = 20
BM = 2048
OUT_W = 128
_MM_DTYPE = jnp.bfloat16


def _body(xst_ref, xb_ref, act_ref, wst_ref, wbt_ref, b1_ref, w2t_ref,
          b2_ref, lp_ref, val_ref, ent_ref):
    bf = _MM_DTYPE
    acc = jnp.dot(wst_ref[...].astype(bf), xst_ref[...].astype(bf),
                  preferred_element_type=jnp.float32)
    acc = acc + jax.lax.dot_general(
        wbt_ref[...].astype(bf), xb_ref[...].astype(bf),
        dimension_numbers=(((1,), (1,)), ((), ())),
        preferred_element_type=jnp.float32)
    ht = jnp.tanh(acc + b1_ref[...])
    o2t = jnp.dot(w2t_ref[...].astype(bf), ht.astype(bf),
                  preferred_element_type=jnp.float32) + b2_ref[...]
    logits = o2t[:A]
    value = o2t[A:A + 1]
    m = jnp.max(logits, axis=0, keepdims=True)
    e = jnp.exp(logits - m)
    z = jnp.sum(e, axis=0, keepdims=True)
    logz = jnp.log(z)
    s = jnp.sum(e * (logits - m), axis=0, keepdims=True)
    ent = logz - s / z
    onehot = jax.lax.broadcasted_iota(jnp.int32, logits.shape, 0) == act_ref[...]
    g = jnp.sum(jnp.where(onehot, logits, 0.0), axis=0, keepdims=True)
    alp = g - m - logz
    lp_ref[...] = alp
    val_ref[...] = value
    ent_ref[...] = ent


@functools.partial(jax.jit, static_argnames=("interpret",))
def _run(states, believes, actions, Wa1, ba1, Wa2, ba2, Wc1, bc1, Wc2, bc2,
         interpret=False):
    wst = jnp.concatenate([Wa1[:DS], Wc1], axis=1).T
    wbt = Wa1[DS:].T
    wbt = jnp.concatenate([wbt, jnp.zeros((H, DB), jnp.float32)], axis=0)
    b1 = jnp.concatenate([ba1, bc1])[:, None]
    w2t = jnp.zeros((OUT_W, 2 * H), jnp.float32)
    w2t = w2t.at[:A, :H].set(Wa2.T).at[A, H:].set(Wc2[:, 0])
    b2 = jnp.zeros((OUT_W, 1), jnp.float32).at[:A, 0].set(ba2).at[A, 0].set(bc2[0])
    statest = states.T
    act2d = actions.astype(jnp.int32)[None, :]

    grid = (B // BM,)
    out = pl.pallas_call(
        _body,
        grid=grid,
        in_specs=[
            pl.BlockSpec((DS, BM), lambda i: (0, i)),
            pl.BlockSpec((BM, DB), lambda i: (i, 0)),
            pl.BlockSpec((1, BM), lambda i: (0, i)),
            pl.BlockSpec((2 * H, DS), lambda i: (0, 0)),
            pl.BlockSpec((2 * H, DB), lambda i: (0, 0)),
            pl.BlockSpec((2 * H, 1), lambda i: (0, 0)),
            pl.BlockSpec((OUT_W, 2 * H), lambda i: (0, 0)),
            pl.BlockSpec((OUT_W, 1), lambda i: (0, 0)),
        ],
        out_specs=[
            pl.BlockSpec((1, BM), lambda i: (0, i)),
            pl.BlockSpec((1, BM), lambda i: (0, i)),
            pl.BlockSpec((1, BM), lambda i: (0, i)),
        ],
        out_shape=[jax.ShapeDtypeStruct((1, B), jnp.float32)] * 3,
        compiler_params=pltpu.CompilerParams(
            dimension_semantics=("parallel",)),
        interpret=interpret,
    )(statest, believes, act2d, wst, wbt, b1, w2t, b2)
    return out[0][0], out[1][0], out[2][0]


def kernel(states, believes, actions, Wa1, ba1, Wa2, ba2, Wc1, bc1, Wc2, bc2):
    return _run(states, believes, actions, Wa1, ba1, Wa2, ba2,
                Wc1, bc1, Wc2, bc2)


# speedup vs baseline: 4.2145x; 1.0174x over previous
import functools

import jax
import jax.numpy as jnp
from jax.experimental import pallas as pl
from jax.experimental.pallas import tpu as pltpu

B = 32768
DS = 658
DB = 250
H = 256
A = 20
BM = 4096
OUT_W = 128
_MM_DTYPE = jnp.bfloat16


def _body(xst_ref, xb_ref, act_ref, wst_ref, wbt_ref, b1_ref, w2t_ref,
          b2_ref, lp_ref, val_ref, ent_ref):
    bf = _MM_DTYPE
    acc = jnp.dot(wst_ref[...].astype(bf), xst_ref[...].astype(bf),
                  preferred_element_type=jnp.float32)
    acc = acc + jax.lax.dot_general(
        wbt_ref[...].astype(bf), xb_ref[...].astype(bf),
        dimension_numbers=(((1,), (1,)), ((), ())),
        preferred_element_type=jnp.float32)
    ht = jnp.tanh(acc + b1_ref[...])
    o2t = jnp.dot(w2t_ref[...].astype(bf), ht.astype(bf),
                  preferred_element_type=jnp.float32) + b2_ref[...]
    logits = o2t[:A]
    value = o2t[A:A + 1]
    m = jnp.max(logits, axis=0, keepdims=True)
    e = jnp.exp(logits - m)
    z = jnp.sum(e, axis=0, keepdims=True)
    logz = jnp.log(z)
    s = jnp.sum(e * (logits - m), axis=0, keepdims=True)
    ent = logz - s / z
    onehot = jax.lax.broadcasted_iota(jnp.int32, logits.shape, 0) == act_ref[...]
    g = jnp.sum(jnp.where(onehot, logits, 0.0), axis=0, keepdims=True)
    alp = g - m - logz
    lp_ref[...] = alp
    val_ref[...] = value
    ent_ref[...] = ent


@functools.partial(jax.jit, static_argnames=("interpret",))
def _run(states, believes, actions, Wa1, ba1, Wa2, ba2, Wc1, bc1, Wc2, bc2,
         interpret=False):
    wst = jnp.concatenate([Wa1[:DS], Wc1], axis=1).T
    wbt = Wa1[DS:].T
    wbt = jnp.concatenate([wbt, jnp.zeros((H, DB), jnp.float32)], axis=0)
    b1 = jnp.concatenate([ba1, bc1])[:, None]
    w2t = jnp.zeros((OUT_W, 2 * H), jnp.float32)
    w2t = w2t.at[:A, :H].set(Wa2.T).at[A, H:].set(Wc2[:, 0])
    b2 = jnp.zeros((OUT_W, 1), jnp.float32).at[:A, 0].set(ba2).at[A, 0].set(bc2[0])
    statest = states.T
    act2d = actions.astype(jnp.int32)[None, :]

    grid = (B // BM,)
    out = pl.pallas_call(
        _body,
        grid=grid,
        in_specs=[
            pl.BlockSpec((DS, BM), lambda i: (0, i)),
            pl.BlockSpec((BM, DB), lambda i: (i, 0)),
            pl.BlockSpec((1, BM), lambda i: (0, i)),
            pl.BlockSpec((2 * H, DS), lambda i: (0, 0)),
            pl.BlockSpec((2 * H, DB), lambda i: (0, 0)),
            pl.BlockSpec((2 * H, 1), lambda i: (0, 0)),
            pl.BlockSpec((OUT_W, 2 * H), lambda i: (0, 0)),
            pl.BlockSpec((OUT_W, 1), lambda i: (0, 0)),
        ],
        out_specs=[
            pl.BlockSpec((1, BM), lambda i: (0, i)),
            pl.BlockSpec((1, BM), lambda i: (0, i)),
            pl.BlockSpec((1, BM), lambda i: (0, i)),
        ],
        out_shape=[jax.ShapeDtypeStruct((1, B), jnp.float32)] * 3,
        compiler_params=pltpu.CompilerParams(
            dimension_semantics=("parallel",)),
        interpret=interpret,
    )(statest, believes, act2d, wst, wbt, b1, w2t, b2)
    return out[0][0], out[1][0], out[2][0]


def kernel(states, believes, actions, Wa1, ba1, Wa2, ba2, Wc1, bc1, Wc2, bc2):
    return _run(states, believes, actions, Wa1, ba1, Wa2, ba2,
                Wc1, bc1, Wc2, bc2)
